# baseline (speedup 1.0000x reference)
import jax
import jax.numpy as jnp
from jax.experimental import pallas as pl
from jax.experimental.pallas import tpu as pltpu


_BLOCK = 8000


def _phi_body(e_hbm, src_ref, tgt_ref, out_ref, e_v, sems):
    i = pl.program_id(0)
    n = pl.num_programs(0)
    slot = jax.lax.rem(i, 2)
    nxt = jax.lax.rem(i + 1, 2)

    @pl.when(i == 0)
    def _prime():
        pltpu.make_async_copy(
            e_hbm.at[pl.ds(0, _BLOCK), :], e_v.at[0], sems.at[0]
        ).start()

    @pl.when(i + 1 < n)
    def _prefetch():
        pltpu.make_async_copy(
            e_hbm.at[pl.ds((i + 1) * _BLOCK, _BLOCK), :], e_v.at[nxt], sems.at[nxt]
        ).start()

    pltpu.make_async_copy(
        e_hbm.at[pl.ds(i * _BLOCK, _BLOCK), :], e_v.at[slot], sems.at[slot]
    ).wait()

    e_blk = e_v[slot]
    de = e_blk.shape[1]
    d = src_ref.shape[1]
    m1 = jnp.full((de, de), 1.0 / de, jnp.float32)
    g16 = jax.nn.sigmoid(jnp.dot(e_blk, m1, preferred_element_type=jnp.float32))
    m2 = jnp.full((de, d), 1.0 / de, jnp.float32)
    gate = jnp.dot(g16, m2, preferred_element_type=jnp.float32)
    out_ref[...] = src_ref[...] * gate + tgt_ref[...]


def kernel(src, e, tgt):
    n, d = src.shape
    de = e.shape[1]
    grid = n // _BLOCK
    return pl.pallas_call(
        _phi_body,
        grid=(grid,),
        in_specs=[
            pl.BlockSpec(memory_space=pl.ANY),
            pl.BlockSpec((_BLOCK, d), lambda i: (i, 0)),
            pl.BlockSpec((_BLOCK, d), lambda i: (i, 0)),
        ],
        out_specs=pl.BlockSpec((_BLOCK, d), lambda i: (i, 0)),
        out_shape=jax.ShapeDtypeStruct((n, d), src.dtype),
        scratch_shapes=[
            pltpu.VMEM((2, _BLOCK, de), jnp.float32),
            pltpu.SemaphoreType.DMA((2,)),
        ],
    )(e, src, tgt)


# speedup vs baseline: 1.7602x; 1.7602x over previous
import jax
import jax.numpy as jnp
from jax import lax
from jax.experimental import pallas as pl


_BLOCK = 6400


def _phi_body(src_ref, et_ref, tgt_ref, out_ref):
    de = et_ref.shape[0]
    d = src_ref.shape[1]
    ones = jnp.full((de, d), 1.0 / de, jnp.float32)
    s = lax.dot_general(
        et_ref[...], ones, (((0,), (0,)), ((), ())),
        preferred_element_type=jnp.float32,
    )
    gate = jax.nn.sigmoid(s)
    out_ref[...] = src_ref[...] * gate + tgt_ref[...]


def kernel(src, e, tgt):
    n, d = src.shape
    de = e.shape[1]
    grid = n // _BLOCK
    return pl.pallas_call(
        _phi_body,
        grid=(grid,),
        in_specs=[
            pl.BlockSpec((_BLOCK, d), lambda i: (i, 0)),
            pl.BlockSpec((de, _BLOCK), lambda i: (0, i)),
            pl.BlockSpec((_BLOCK, d), lambda i: (i, 0)),
        ],
        out_specs=pl.BlockSpec((_BLOCK, d), lambda i: (i, 0)),
        out_shape=jax.ShapeDtypeStruct((n, d), src.dtype),
    )(src, e.T, tgt)


# speedup vs baseline: 1.7994x; 1.0222x over previous
import jax
import jax.numpy as jnp
from jax import lax
from jax.experimental import pallas as pl


_BLOCK = 12800


def _phi_body(src_ref, et_ref, tgt_ref, out_ref):
    de = et_ref.shape[0]
    d = src_ref.shape[1]
    ones = jnp.full((de, d), 1.0 / de, jnp.float32)
    s = lax.dot_general(
        et_ref[...], ones, (((0,), (0,)), ((), ())),
        preferred_element_type=jnp.float32,
    )
    gate = jax.nn.sigmoid(s)
    out_ref[...] = src_ref[...] * gate + tgt_ref[...]


def kernel(src, e, tgt):
    n, d = src.shape
    de = e.shape[1]
    grid = n // _BLOCK
    return pl.pallas_call(
        _phi_body,
        grid=(grid,),
        in_specs=[
            pl.BlockSpec((_BLOCK, d), lambda i: (i, 0)),
            pl.BlockSpec((de, _BLOCK), lambda i: (0, i)),
            pl.BlockSpec((_BLOCK, d), lambda i: (i, 0)),
        ],
        out_specs=pl.BlockSpec((_BLOCK, d), lambda i: (i, 0)),
        out_shape=jax.ShapeDtypeStruct((n, d), src.dtype),
    )(src, e.T, tgt)
